# Initial kernel scaffold; baseline (speedup 1.0000x reference)
import jax
import jax.numpy as jnp
from jax.experimental import pallas as pl


def kernel(images, box_preds, cls_preds, anchors):
    raise NotImplementedError("write your pallas kernel here")


# speedup vs baseline: 3.6915x; 3.6915x over previous
import jax
import jax.numpy as jnp
from jax import lax
from jax.experimental import pallas as pl

_IOU = 0.5
_CONF = 0.5
_MAXDET = 100
_N = 20000
_R = 160
_C = 128
_NP = _R * _C
_NCLS = 20
_B = 2


def _nms_body(cls_ref, box_ref, anc_ref, out_ref):
    best_s = jax.nn.sigmoid(cls_ref[:, 0])
    best_c = jnp.zeros((_B, _R, _C), jnp.float32)
    for c in range(1, _NCLS):
        s = jax.nn.sigmoid(cls_ref[:, c])
        m = s > best_s
        best_s = jnp.where(m, s, best_s)
        best_c = jnp.where(m, jnp.float32(c), best_c)
    conf = best_s
    scores0 = jnp.where(conf > _CONF, conf, -1.0)

    ax = anc_ref[0][None]
    ay = anc_ref[1][None]
    aw = anc_ref[2][None]
    ah = anc_ref[3][None]
    d0 = box_ref[:, 0] * jnp.float32(0.1)
    d1 = box_ref[:, 1] * jnp.float32(0.1)
    d2 = box_ref[:, 2] * jnp.float32(0.2)
    d3 = box_ref[:, 3] * jnp.float32(0.2)
    bx = d0 * aw + ax
    by = d1 * ah + ay
    bw = jnp.exp(d2) * aw
    bh = jnp.exp(d3) * ah
    off = best_c * jnp.float32(10000.0)
    x1 = bx + off
    y1 = by + off
    x2 = (bx + bw) + off
    y2 = (by + bh) + off
    areas = (x2 - x1) * (y2 - y1)

    iota3 = (lax.broadcasted_iota(jnp.int32, (1, _R, _C), 1) * _C
             + lax.broadcasted_iota(jnp.int32, (1, _R, _C), 2))
    lane3 = lax.broadcasted_iota(jnp.int32, (1, 1, _C), 2)
    zrow = jnp.zeros((_B, 1, _C), jnp.float32)

    def bmax(a):
        return jnp.max(jnp.max(a, axis=2), axis=1).reshape(_B, 1, 1)

    def bmin(a):
        return jnp.min(jnp.min(a, axis=2), axis=1).reshape(_B, 1, 1)

    def body(i, st):
        sc, obx, oby, obw, obh, ocl, ocf, ovl = st
        bv = bmax(sc)
        bidx = bmin(jnp.where(sc == bv, iota3, jnp.int32(2 ** 30)))
        m = iota3 == bidx

        def ext(a):
            return jnp.sum(jnp.sum(jnp.where(m, a, 0.0), axis=2),
                           axis=1).reshape(_B, 1, 1)

        ex1 = ext(x1)
        ey1 = ext(y1)
        ex2 = ext(x2)
        ey2 = ext(y2)
        ear = ext(areas)
        xx1 = jnp.maximum(ex1, x1)
        yy1 = jnp.maximum(ey1, y1)
        xx2 = jnp.minimum(ex2, x2)
        yy2 = jnp.minimum(ey2, y2)
        inter = jnp.maximum(xx2 - xx1, 0.0) * jnp.maximum(yy2 - yy1, 0.0)
        iou = inter / (ear + areas - inter + jnp.float32(1e-8))
        sc = jnp.where(iou >= _IOU, -1.0, sc)

        valid = jnp.where(bv > 0.0, 1.0, 0.0)
        sel = lane3 == i
        obx = jnp.where(sel, ext(bx), obx)
        oby = jnp.where(sel, ext(by), oby)
        obw = jnp.where(sel, ext(bw), obw)
        obh = jnp.where(sel, ext(bh), obh)
        ocl = jnp.where(sel, ext(best_c), ocl)
        ocf = jnp.where(sel, ext(conf), ocf)
        ovl = jnp.where(sel, valid, ovl)
        return sc, obx, oby, obw, obh, ocl, ocf, ovl

    st0 = (scores0, zrow, zrow, zrow, zrow, zrow, zrow, zrow)
    _, obx, oby, obw, obh, ocl, ocf, ovl = lax.fori_loop(0, _MAXDET, body, st0)

    vm = ovl > 0.0
    neg = jnp.full((_B, 1, _C), -1.0, jnp.float32)
    out_ref[...] = jnp.concatenate([
        jnp.where(vm, obx, neg),
        jnp.where(vm, oby, neg),
        jnp.where(vm, obw, neg),
        jnp.where(vm, obh, neg),
        jnp.where(vm, ocl, neg),
        jnp.where(vm, ocf, neg),
        ovl,
        zrow,
    ], axis=1)


def kernel(images, box_preds, cls_preds, anchors):
    del images
    B = box_preds.shape[0]
    padn = _NP - _N
    cls_t = jnp.transpose(cls_preds, (0, 2, 1))
    cls_t = jnp.pad(cls_t, ((0, 0), (0, 0), (0, padn)), constant_values=-1e9)
    cls_t = cls_t.reshape(B, _NCLS, _R, _C)
    box_t = jnp.transpose(box_preds, (0, 2, 1))
    box_t = jnp.pad(box_t, ((0, 0), (0, 0), (0, padn))).reshape(B, 4, _R, _C)
    anc_t = jnp.pad(anchors.T, ((0, 0), (0, padn))).reshape(4, _R, _C)

    out = pl.pallas_call(
        _nms_body,
        out_shape=jax.ShapeDtypeStruct((B, 8, _C), jnp.float32),
    )(cls_t, box_t, anc_t)

    out_boxes = jnp.stack(
        [out[:, 0, :_MAXDET], out[:, 1, :_MAXDET],
         out[:, 2, :_MAXDET], out[:, 3, :_MAXDET]], axis=-1)
    out_classes = out[:, 4, :_MAXDET]
    out_conf = out[:, 5, :_MAXDET]
    return out_boxes, out_classes, out_conf


# speedup vs baseline: 4.4692x; 1.2107x over previous
import jax
import jax.numpy as jnp
from jax import lax
from jax.experimental import pallas as pl
from jax.experimental.pallas import tpu as pltpu

_IOU = 0.5
_CONF = 0.5
_MAXDET = 100
_N = 20000
_R = 160
_C = 128
_NP = _R * _C
_NCLS = 20


def _nms_body(cls_ref, box_ref, anc_ref, out_ref, sc_ref, stk_ref):
    best_s = jax.nn.sigmoid(cls_ref[0, 0])
    best_c = jnp.zeros((_R, _C), jnp.float32)
    for c in range(1, _NCLS):
        s = jax.nn.sigmoid(cls_ref[0, c])
        m = s > best_s
        best_s = jnp.where(m, s, best_s)
        best_c = jnp.where(m, jnp.float32(c), best_c)
    conf = best_s
    scores0 = jnp.where(conf > _CONF, conf, -1.0)

    ax = anc_ref[0]
    ay = anc_ref[1]
    aw = anc_ref[2]
    ah = anc_ref[3]
    d0 = box_ref[0, 0] * jnp.float32(0.1)
    d1 = box_ref[0, 1] * jnp.float32(0.1)
    d2 = box_ref[0, 2] * jnp.float32(0.2)
    d3 = box_ref[0, 3] * jnp.float32(0.2)
    bx = d0 * aw + ax
    by = d1 * ah + ay
    bw = jnp.exp(d2) * aw
    bh = jnp.exp(d3) * ah
    off = best_c * jnp.float32(10000.0)
    x1 = bx + off
    y1 = by + off
    x2 = (bx + bw) + off
    y2 = (by + bh) + off
    areas = (x2 - x1) * (y2 - y1)

    stk_ref[...] = jnp.stack(
        [x1, y1, x2, y2, areas, bx, by, bw, bh, best_c, conf],
        axis=1)
    sc_ref[...] = scores0

    iota_r = lax.broadcasted_iota(jnp.int32, (_R, 1), 0)
    lane = lax.broadcasted_iota(jnp.int32, (1, _C), 1)
    zrow = jnp.zeros((1, _C), jnp.float32)
    big = jnp.int32(2 ** 30)

    def body(i, st):
        obx, oby, obw, obh, ocl, ocf, ovl = st
        sc = sc_ref[...]
        rm = jnp.max(sc, axis=1, keepdims=True)
        bv = jnp.max(rm)
        r = jnp.min(jnp.where(rm == bv, iota_r, big))
        srow = sc_ref[pl.ds(r, 1), :]
        c = jnp.min(jnp.where(srow == bv, lane, big))

        vrow = stk_ref[pl.ds(r, 1), :, :].reshape(11, _C)
        vals = jnp.sum(jnp.where(lane == c, vrow, 0.0), axis=1,
                       keepdims=True)
        ex1 = vals[0:1]
        ey1 = vals[1:2]
        ex2 = vals[2:3]
        ey2 = vals[3:4]
        ear = vals[4:5]
        xx1 = jnp.maximum(ex1, x1)
        yy1 = jnp.maximum(ey1, y1)
        xx2 = jnp.minimum(ex2, x2)
        yy2 = jnp.minimum(ey2, y2)
        inter = jnp.maximum(xx2 - xx1, 0.0) * jnp.maximum(yy2 - yy1, 0.0)
        iou = inter / (ear + areas - inter + jnp.float32(1e-8))
        sc_ref[...] = jnp.where(iou >= _IOU, -1.0, sc)

        valid = jnp.where(bv > 0.0, 1.0, 0.0)
        sel = lane == i
        obx = jnp.where(sel, vals[5:6], obx)
        oby = jnp.where(sel, vals[6:7], oby)
        obw = jnp.where(sel, vals[7:8], obw)
        obh = jnp.where(sel, vals[8:9], obh)
        ocl = jnp.where(sel, vals[9:10], ocl)
        ocf = jnp.where(sel, vals[10:11], ocf)
        ovl = jnp.where(sel, valid, ovl)
        return obx, oby, obw, obh, ocl, ocf, ovl

    st0 = (zrow, zrow, zrow, zrow, zrow, zrow, zrow)
    obx, oby, obw, obh, ocl, ocf, ovl = lax.fori_loop(0, _MAXDET, body, st0)

    vm = ovl > 0.0
    neg = jnp.full((1, _C), -1.0, jnp.float32)
    out_ref[0] = jnp.concatenate([
        jnp.where(vm, obx, neg),
        jnp.where(vm, oby, neg),
        jnp.where(vm, obw, neg),
        jnp.where(vm, obh, neg),
        jnp.where(vm, ocl, neg),
        jnp.where(vm, ocf, neg),
        ovl,
        zrow,
    ], axis=0)


def kernel(images, box_preds, cls_preds, anchors):
    del images
    B = box_preds.shape[0]
    padn = _NP - _N
    cls_t = jnp.transpose(cls_preds, (0, 2, 1))
    cls_t = jnp.pad(cls_t, ((0, 0), (0, 0), (0, padn)), constant_values=-1e9)
    cls_t = cls_t.reshape(B, _NCLS, _R, _C)
    box_t = jnp.transpose(box_preds, (0, 2, 1))
    box_t = jnp.pad(box_t, ((0, 0), (0, 0), (0, padn))).reshape(B, 4, _R, _C)
    anc_t = jnp.pad(anchors.T, ((0, 0), (0, padn))).reshape(4, _R, _C)

    out = pl.pallas_call(
        _nms_body,
        grid=(B,),
        in_specs=[
            pl.BlockSpec((1, _NCLS, _R, _C), lambda b: (b, 0, 0, 0)),
            pl.BlockSpec((1, 4, _R, _C), lambda b: (b, 0, 0, 0)),
            pl.BlockSpec((4, _R, _C), lambda b: (0, 0, 0)),
        ],
        out_specs=pl.BlockSpec((1, 8, _C), lambda b: (b, 0, 0)),
        out_shape=jax.ShapeDtypeStruct((B, 8, _C), jnp.float32),
        scratch_shapes=[
            pltpu.VMEM((_R, _C), jnp.float32),
            pltpu.VMEM((_R, 11, _C), jnp.float32),
        ],
    )(cls_t, box_t, anc_t)

    out_boxes = jnp.stack(
        [out[:, 0, :_MAXDET], out[:, 1, :_MAXDET],
         out[:, 2, :_MAXDET], out[:, 3, :_MAXDET]], axis=-1)
    out_classes = out[:, 4, :_MAXDET]
    out_conf = out[:, 5, :_MAXDET]
    return out_boxes, out_classes, out_conf


# speedup vs baseline: 6.2850x; 1.4063x over previous
import jax
import jax.numpy as jnp
from jax import lax
from jax.experimental import pallas as pl
from jax.experimental.pallas import tpu as pltpu

_IOU = 0.5
_CONF = 0.5
_MAXDET = 100
_N = 20000
_R = 160
_C = 128
_NP = _R * _C
_NCLS = 20


def _decode_one(cls_ref, box_ref, anc_ref, b, sc_ref, stk_ref):
    best_s = jax.nn.sigmoid(cls_ref[b, 0])
    best_c = jnp.zeros((_R, _C), jnp.float32)
    for c in range(1, _NCLS):
        s = jax.nn.sigmoid(cls_ref[b, c])
        m = s > best_s
        best_s = jnp.where(m, s, best_s)
        best_c = jnp.where(m, jnp.float32(c), best_c)
    conf = best_s
    ax = anc_ref[0]
    ay = anc_ref[1]
    aw = anc_ref[2]
    ah = anc_ref[3]
    d0 = box_ref[b, 0] * jnp.float32(0.1)
    d1 = box_ref[b, 1] * jnp.float32(0.1)
    d2 = box_ref[b, 2] * jnp.float32(0.2)
    d3 = box_ref[b, 3] * jnp.float32(0.2)
    bx = d0 * aw + ax
    by = d1 * ah + ay
    bw = jnp.exp(d2) * aw
    bh = jnp.exp(d3) * ah
    off = best_c * jnp.float32(10000.0)
    x1 = bx + off
    y1 = by + off
    x2 = (bx + bw) + off
    y2 = (by + bh) + off
    areas = (x2 - x1) * (y2 - y1)
    stk_ref[...] = jnp.stack(
        [x1, y1, x2, y2, areas, bx, by, bw, bh, best_c, conf], axis=1)
    sc_ref[...] = jnp.where(conf > _CONF, conf, -1.0)
    return x1, y1, x2, y2, areas


def _nms_body(cls_ref, box_ref, anc_ref, out_ref,
              sc0_ref, stk0_ref, sc1_ref, stk1_ref):
    geo0 = _decode_one(cls_ref, box_ref, anc_ref, 0, sc0_ref, stk0_ref)
    geo1 = _decode_one(cls_ref, box_ref, anc_ref, 1, sc1_ref, stk1_ref)

    iota2 = (lax.broadcasted_iota(jnp.int32, (_R, _C), 0) * _C
             + lax.broadcasted_iota(jnp.int32, (_R, _C), 1))
    lane = lax.broadcasted_iota(jnp.int32, (1, _C), 1)
    zrow = jnp.zeros((1, _C), jnp.float32)
    big = jnp.int32(2 ** 30)

    def step(i, sc_ref, stk_ref, geo, st):
        obx, oby, obw, obh, ocl, ocf, ovl = st
        x1, y1, x2, y2, areas = geo
        sc = sc_ref[...]
        bv = jnp.max(sc)
        bidx = jnp.min(jnp.where(sc == bv, iota2, big))
        r = lax.shift_right_logical(bidx, 7)
        c = jnp.bitwise_and(bidx, jnp.int32(_C - 1))

        vrow = stk_ref[pl.ds(r, 1), :, :].reshape(11, _C)
        vals = jnp.sum(jnp.where(lane == c, vrow, 0.0), axis=1,
                       keepdims=True)
        xx1 = jnp.maximum(vals[0:1], x1)
        yy1 = jnp.maximum(vals[1:2], y1)
        xx2 = jnp.minimum(vals[2:3], x2)
        yy2 = jnp.minimum(vals[3:4], y2)
        inter = jnp.maximum(xx2 - xx1, 0.0) * jnp.maximum(yy2 - yy1, 0.0)
        iou = inter / (vals[4:5] + areas - inter + jnp.float32(1e-8))
        sc_ref[...] = jnp.where(iou >= _IOU, -1.0, sc)

        valid = jnp.where(bv > 0.0, 1.0, 0.0)
        sel = lane == i
        obx = jnp.where(sel, vals[5:6], obx)
        oby = jnp.where(sel, vals[6:7], oby)
        obw = jnp.where(sel, vals[7:8], obw)
        obh = jnp.where(sel, vals[8:9], obh)
        ocl = jnp.where(sel, vals[9:10], ocl)
        ocf = jnp.where(sel, vals[10:11], ocf)
        ovl = jnp.where(sel, valid, ovl)
        return obx, oby, obw, obh, ocl, ocf, ovl

    def body(i, st):
        st0, st1 = st[:7], st[7:]
        st0 = step(i, sc0_ref, stk0_ref, geo0, st0)
        st1 = step(i, sc1_ref, stk1_ref, geo1, st1)
        return st0 + st1

    init = (zrow,) * 14
    fin = lax.fori_loop(0, _MAXDET, body, init)

    neg = jnp.full((1, _C), -1.0, jnp.float32)
    for b, stb in ((0, fin[:7]), (1, fin[7:])):
        obx, oby, obw, obh, ocl, ocf, ovl = stb
        vm = ovl > 0.0
        out_ref[b] = jnp.concatenate([
            jnp.where(vm, obx, neg),
            jnp.where(vm, oby, neg),
            jnp.where(vm, obw, neg),
            jnp.where(vm, obh, neg),
            jnp.where(vm, ocl, neg),
            jnp.where(vm, ocf, neg),
            ovl,
            zrow,
        ], axis=0)


def kernel(images, box_preds, cls_preds, anchors):
    del images
    B = box_preds.shape[0]
    padn = _NP - _N
    cls_t = jnp.transpose(cls_preds, (0, 2, 1))
    cls_t = jnp.pad(cls_t, ((0, 0), (0, 0), (0, padn)), constant_values=-1e9)
    cls_t = cls_t.reshape(B, _NCLS, _R, _C)
    box_t = jnp.transpose(box_preds, (0, 2, 1))
    box_t = jnp.pad(box_t, ((0, 0), (0, 0), (0, padn))).reshape(B, 4, _R, _C)
    anc_t = jnp.pad(anchors.T, ((0, 0), (0, padn))).reshape(4, _R, _C)

    out = pl.pallas_call(
        _nms_body,
        out_shape=jax.ShapeDtypeStruct((B, 8, _C), jnp.float32),
        scratch_shapes=[
            pltpu.VMEM((_R, _C), jnp.float32),
            pltpu.VMEM((_R, 11, _C), jnp.float32),
            pltpu.VMEM((_R, _C), jnp.float32),
            pltpu.VMEM((_R, 11, _C), jnp.float32),
        ],
    )(cls_t, box_t, anc_t)

    out_boxes = jnp.stack(
        [out[:, 0, :_MAXDET], out[:, 1, :_MAXDET],
         out[:, 2, :_MAXDET], out[:, 3, :_MAXDET]], axis=-1)
    out_classes = out[:, 4, :_MAXDET]
    out_conf = out[:, 5, :_MAXDET]
    return out_boxes, out_classes, out_conf
